# baseline (speedup 1.0000x reference)
import functools

import jax
import jax.numpy as jnp
from jax import lax
from jax.experimental import pallas as pl
from jax.experimental.pallas import tpu as pltpu
from jax.experimental.pallas import tpu_sc as plsc

N = 10000
M = 32
F_ATOM = 128
F_BOND = 16

NC = 2
NS = 16
NW = NC * NS
N_PAD = 10240
CPW = N_PAD // NW

_LANES = 16
_FBLKS = F_ATOM // _LANES

def _prep_body(bond_ref, atom_ref, w_ref, tab_ref):
    x = bond_ref[0]
    s = jnp.sum(x * x, axis=-1)
    w = 1.0 / s
    d = jnp.maximum(jnp.sum(jnp.abs(w), axis=-1, keepdims=True), 1e-12)
    w_ref[...] = w / d
    a = atom_ref[...]
    xl = jnp.concatenate([a[:, b * 32:b * 32 + 16] for b in range(4)], axis=1)
    xh = jnp.concatenate([a[:, b * 32 + 16:b * 32 + 32] for b in range(4)],
                         axis=1)
    lo = jax.lax.bitcast_convert_type(
        xl.astype(jnp.bfloat16), jnp.uint16).astype(jnp.uint32)
    hi = jax.lax.bitcast_convert_type(
        xh.astype(jnp.bfloat16), jnp.uint16).astype(jnp.uint32)
    tab_ref[...] = jax.lax.bitcast_convert_type((hi << 16) | lo, jnp.int32)


def _prep(bond4, atom2):
    nb = 1000
    grid = N // nb
    return pl.pallas_call(
        _prep_body,
        grid=(grid,),
        in_specs=[
            pl.BlockSpec((1, nb, M, F_BOND), lambda i: (0, i, 0, 0)),
            pl.BlockSpec((nb, F_ATOM), lambda i: (i, 0)),
        ],
        out_specs=[
            pl.BlockSpec((nb, M), lambda i: (i, 0)),
            pl.BlockSpec((nb, _TW), lambda i: (i, 0)),
        ],
        out_shape=[
            jax.ShapeDtypeStruct((N, M), jnp.float32),
            jax.ShapeDtypeStruct((N_PAD, _TW), jnp.int32),
        ],
    )(bond4, atom2)


G = 4
NG = CPW // G
_GI = G * M
_TW = F_ATOM // 2

_STAGE = N_PAD // NS
CH = 16
_GPC = CH // G
_NCH = CPW // CH


def _sc_body(tab_hbm, adj_hbm, w_hbm, out_hbm,
             tab_sh, idx_v, w_v, och0, och1, rows,
             sem0, sem1, osem0, osem1):
    c = lax.axis_index("c")
    s = lax.axis_index("s")
    wid = s * NC + c

    pltpu.sync_copy(tab_hbm.at[pl.ds(s * _STAGE, _STAGE)],
                    tab_sh.at[pl.ds(s * _STAGE, _STAGE)])
    pltpu.sync_copy(adj_hbm.at[wid], idx_v)
    pltpu.sync_copy(w_hbm.at[wid], w_v)
    plsc.subcore_barrier()

    def issue(g, half, sem):
        pltpu.async_copy(tab_sh.at[idx_v.at[g]],
                         rows.at[pl.ds(half * _GI, _GI)], sem)

    def wait(g, half, sem):
        pltpu.make_async_copy(tab_sh.at[idx_v.at[g]],
                              rows.at[pl.ds(half * _GI, _GI)], sem).wait()

    def out_issue(ch, buf, osem):
        pltpu.async_copy(buf, out_hbm.at[wid, pl.ds(ch * CH, CH)], osem)

    def out_wait(buf, osem):
        pltpu.make_async_copy(buf, out_hbm.at[wid, pl.ds(0, CH)], osem).wait()

    issue(0, 0, sem0)
    issue(1, 1, sem1)

    def body(g, carry):
        p = lax.rem(g, 2)
        ch = lax.div(g, _GPC)
        gc = lax.rem(g, _GPC)
        chp = lax.rem(ch, 2)

        @pl.when(jnp.logical_and(gc == 0, ch >= 2))
        def _():
            @pl.when(chp == 0)
            def _():
                out_wait(och0, osem0)

            @pl.when(chp == 1)
            def _():
                out_wait(och1, osem1)

        @pl.when(p == 0)
        def _():
            wait(g, 0, sem0)

        @pl.when(p == 1)
        def _():
            wait(g, 1, sem1)

        off = p * _GI
        for j in range(G):
            node = g * G + j
            crow = gc * G + j
            wrows = [w_v[node, pl.ds(h * _LANES, _LANES)]
                     for h in range(M // _LANES)]
            ws = [wrows[m // _LANES][m % _LANES] for m in range(M)]
            for b in range(4):
                sl = pl.ds(b * _LANES, _LANES)
                e0, o0 = plsc.unpack(
                    plsc.bitcast(rows[off + j * M, sl], jnp.bfloat16),
                    format=plsc.PackFormat.INTERLEAVED)
                acc_e = ws[0] * e0
                acc_o = ws[0] * o0
                for m in range(1, M):
                    e, o = plsc.unpack(
                        plsc.bitcast(rows[off + j * M + m, sl], jnp.bfloat16),
                        format=plsc.PackFormat.INTERLEAVED)
                    acc_e = acc_e + ws[m] * e
                    acc_o = acc_o + ws[m] * o

                @pl.when(chp == 0)
                def _(acc_e=acc_e, acc_o=acc_o, crow=crow, b=b):
                    och0[crow, pl.ds(b * 32, _LANES)] = acc_e
                    och0[crow, pl.ds(b * 32 + _LANES, _LANES)] = acc_o

                @pl.when(chp == 1)
                def _(acc_e=acc_e, acc_o=acc_o, crow=crow, b=b):
                    och1[crow, pl.ds(b * 32, _LANES)] = acc_e
                    och1[crow, pl.ds(b * 32 + _LANES, _LANES)] = acc_o

        @pl.when(jnp.logical_and(p == 0, g + 2 < NG))
        def _():
            issue(g + 2, 0, sem0)

        @pl.when(jnp.logical_and(p == 1, g + 2 < NG))
        def _():
            issue(g + 2, 1, sem1)

        @pl.when(gc == _GPC - 1)
        def _():
            @pl.when(chp == 0)
            def _():
                out_issue(ch, och0, osem0)

            @pl.when(chp == 1)
            def _():
                out_issue(ch, och1, osem1)

        return carry

    lax.fori_loop(0, NG, body, 0)
    out_wait(och0, osem0)
    out_wait(och1, osem1)


def _sc_aggregate(tab, adj3, w3):
    mesh = plsc.VectorSubcoreMesh(core_axis_name="c", subcore_axis_name="s",
                                  num_cores=NC, num_subcores=NS)
    f = pl.kernel(
        _sc_body,
        out_type=jax.ShapeDtypeStruct((NW, CPW, F_ATOM), jnp.float32),
        mesh=mesh,
        compiler_params=pltpu.CompilerParams(needs_layout_passes=False,
                                             use_tc_tiling_on_sc=False),
        scratch_types=[
            pltpu.VMEM_SHARED((N_PAD, _TW), jnp.int32),
            pltpu.VMEM((NG, _GI), jnp.int32),
            pltpu.VMEM((CPW, M), jnp.float32),
            pltpu.VMEM((CH, F_ATOM), jnp.float32),
            pltpu.VMEM((CH, F_ATOM), jnp.float32),
            pltpu.VMEM((2 * _GI, _TW), jnp.int32),
            pltpu.SemaphoreType.DMA,
            pltpu.SemaphoreType.DMA,
            pltpu.SemaphoreType.DMA,
            pltpu.SemaphoreType.DMA,
        ],
    )
    return f(tab, adj3, w3)


def _out_body(atom_ref, agg_ref, w1_ref, b1_ref, out_ref):
    x = atom_ref[...] * agg_ref[...]
    y = jnp.dot(x, w1_ref[...], preferred_element_type=jnp.float32)
    out_ref[...] = jnp.maximum(y + b1_ref[...], 0.0)


def _linear_relu(atom2, agg2, W1, b1):
    nb = 1000
    grid = N // nb
    return pl.pallas_call(
        _out_body,
        grid=(grid,),
        in_specs=[
            pl.BlockSpec((nb, F_ATOM), lambda i: (i, 0)),
            pl.BlockSpec((nb, F_ATOM), lambda i: (i, 0)),
            pl.BlockSpec((F_ATOM, F_ATOM), lambda i: (0, 0)),
            pl.BlockSpec((1, F_ATOM), lambda i: (0, 0)),
        ],
        out_specs=pl.BlockSpec((nb, F_ATOM), lambda i: (i, 0)),
        out_shape=jax.ShapeDtypeStruct((N, F_ATOM), jnp.float32),
    )(atom2, agg2, W1, b1.reshape(1, F_ATOM))


@jax.jit
def kernel(atom, bond, adj_matrix, W1, b1):
    atom2 = atom[0]
    w, tab = _prep(bond, atom2)

    pad = ((0, N_PAD - N), (0, 0))
    adj3 = jnp.pad(adj_matrix[0], pad).reshape(NW, NG, _GI)
    w3 = jnp.pad(w, pad).reshape(NW, CPW, M)

    agg = _sc_aggregate(tab, adj3, w3)
    agg2 = agg.reshape(N_PAD, F_ATOM)[:N]

    out = _linear_relu(atom2, agg2, W1, b1)
    return out.reshape(1, N, F_ATOM)


# speedup vs baseline: 1.9024x; 1.9024x over previous
import functools

import jax
import jax.numpy as jnp
from jax import lax
from jax.experimental import pallas as pl
from jax.experimental.pallas import tpu as pltpu
from jax.experimental.pallas import tpu_sc as plsc

N = 10000
M = 32
F_ATOM = 128
F_BOND = 16

NC = 2
NS = 16
NW = NC * NS
N_PAD = 10240
CPW = N_PAD // NW

_LANES = 16
_FBLKS = F_ATOM // _LANES

def _prep_body(bond_ref, atom_ref, ones_ref, w_ref, tab_ref):
    x = bond_ref[...]
    s = jnp.dot(x * x, ones_ref[...],
                preferred_element_type=jnp.float32)
    w = 1.0 / s
    d = jnp.maximum(jnp.sum(jnp.abs(w), axis=-1, keepdims=True), 1e-12)
    w_ref[...] = w / d
    a = atom_ref[...]
    xl = jnp.concatenate([a[:, b * 32:b * 32 + 16] for b in range(4)], axis=1)
    xh = jnp.concatenate([a[:, b * 32 + 16:b * 32 + 32] for b in range(4)],
                         axis=1)
    lo = jax.lax.bitcast_convert_type(
        xl.astype(jnp.bfloat16), jnp.uint16).astype(jnp.uint32)
    hi = jax.lax.bitcast_convert_type(
        xh.astype(jnp.bfloat16), jnp.uint16).astype(jnp.uint32)
    tab_ref[...] = jax.lax.bitcast_convert_type((hi << 16) | lo, jnp.int32)


def _prep(bond2, atom2):
    nb = 1000
    grid = N // nb
    ones_bd = (jnp.arange(M * F_BOND)[:, None] // F_BOND
               == jnp.arange(M)[None, :]).astype(jnp.float32)
    return pl.pallas_call(
        _prep_body,
        grid=(grid,),
        in_specs=[
            pl.BlockSpec((nb, M * F_BOND), lambda i: (i, 0)),
            pl.BlockSpec((nb, F_ATOM), lambda i: (i, 0)),
            pl.BlockSpec((M * F_BOND, M), lambda i: (0, 0)),
        ],
        out_specs=[
            pl.BlockSpec((nb, M), lambda i: (i, 0)),
            pl.BlockSpec((nb, _TW), lambda i: (i, 0)),
        ],
        out_shape=[
            jax.ShapeDtypeStruct((N, M), jnp.float32),
            jax.ShapeDtypeStruct((N_PAD, _TW), jnp.int32),
        ],
    )(bond2, atom2, ones_bd)


G = 4
NG = CPW // G
_GI = G * M
_TW = F_ATOM // 2

_STAGE = N_PAD // NS
CH = 16
_GPC = CH // G
_NCH = CPW // CH


def _sc_body(tab_hbm, adj_hbm, w_hbm, out_hbm,
             tab_sh, idx_v, w_v, och0, och1, rows,
             sem0, sem1, osem0, osem1):
    c = lax.axis_index("c")
    s = lax.axis_index("s")
    wid = s * NC + c

    pltpu.sync_copy(tab_hbm.at[pl.ds(s * _STAGE, _STAGE)],
                    tab_sh.at[pl.ds(s * _STAGE, _STAGE)])
    pltpu.sync_copy(adj_hbm.at[wid], idx_v)
    pltpu.sync_copy(w_hbm.at[wid], w_v)
    plsc.subcore_barrier()

    def issue(g, half, sem):
        pltpu.async_copy(tab_sh.at[idx_v.at[g]],
                         rows.at[pl.ds(half * _GI, _GI)], sem)

    def wait(g, half, sem):
        pltpu.make_async_copy(tab_sh.at[idx_v.at[g]],
                              rows.at[pl.ds(half * _GI, _GI)], sem).wait()

    def out_issue(ch, buf, osem):
        pltpu.async_copy(buf, out_hbm.at[wid, pl.ds(ch * CH, CH)], osem)

    def out_wait(buf, osem):
        pltpu.make_async_copy(buf, out_hbm.at[wid, pl.ds(0, CH)], osem).wait()

    issue(0, 0, sem0)
    issue(1, 1, sem1)

    def body(g, carry):
        p = lax.rem(g, 2)
        ch = lax.div(g, _GPC)
        gc = lax.rem(g, _GPC)
        chp = lax.rem(ch, 2)

        @pl.when(jnp.logical_and(gc == 0, ch >= 2))
        def _():
            @pl.when(chp == 0)
            def _():
                out_wait(och0, osem0)

            @pl.when(chp == 1)
            def _():
                out_wait(och1, osem1)

        @pl.when(p == 0)
        def _():
            wait(g, 0, sem0)

        @pl.when(p == 1)
        def _():
            wait(g, 1, sem1)

        off = p * _GI
        for j in range(G):
            node = g * G + j
            crow = gc * G + j
            wrows = [w_v[node, pl.ds(h * _LANES, _LANES)]
                     for h in range(M // _LANES)]
            ws = [wrows[m // _LANES][m % _LANES] for m in range(M)]
            for b in range(4):
                sl = pl.ds(b * _LANES, _LANES)
                e0, o0 = plsc.unpack(
                    plsc.bitcast(rows[off + j * M, sl], jnp.bfloat16),
                    format=plsc.PackFormat.INTERLEAVED)
                acc_e = ws[0] * e0
                acc_o = ws[0] * o0
                for m in range(1, M):
                    e, o = plsc.unpack(
                        plsc.bitcast(rows[off + j * M + m, sl], jnp.bfloat16),
                        format=plsc.PackFormat.INTERLEAVED)
                    acc_e = acc_e + ws[m] * e
                    acc_o = acc_o + ws[m] * o

                @pl.when(chp == 0)
                def _(acc_e=acc_e, acc_o=acc_o, crow=crow, b=b):
                    och0[crow, pl.ds(b * 32, _LANES)] = acc_e
                    och0[crow, pl.ds(b * 32 + _LANES, _LANES)] = acc_o

                @pl.when(chp == 1)
                def _(acc_e=acc_e, acc_o=acc_o, crow=crow, b=b):
                    och1[crow, pl.ds(b * 32, _LANES)] = acc_e
                    och1[crow, pl.ds(b * 32 + _LANES, _LANES)] = acc_o

        @pl.when(jnp.logical_and(p == 0, g + 2 < NG))
        def _():
            issue(g + 2, 0, sem0)

        @pl.when(jnp.logical_and(p == 1, g + 2 < NG))
        def _():
            issue(g + 2, 1, sem1)

        @pl.when(gc == _GPC - 1)
        def _():
            @pl.when(chp == 0)
            def _():
                out_issue(ch, och0, osem0)

            @pl.when(chp == 1)
            def _():
                out_issue(ch, och1, osem1)

        return carry

    lax.fori_loop(0, NG, body, 0)
    out_wait(och0, osem0)
    out_wait(och1, osem1)


def _sc_aggregate(tab, adj3, w3):
    mesh = plsc.VectorSubcoreMesh(core_axis_name="c", subcore_axis_name="s",
                                  num_cores=NC, num_subcores=NS)
    f = pl.kernel(
        _sc_body,
        out_type=jax.ShapeDtypeStruct((NW, CPW, F_ATOM), jnp.float32),
        mesh=mesh,
        compiler_params=pltpu.CompilerParams(needs_layout_passes=False,
                                             use_tc_tiling_on_sc=False),
        scratch_types=[
            pltpu.VMEM_SHARED((N_PAD, _TW), jnp.int32),
            pltpu.VMEM((NG, _GI), jnp.int32),
            pltpu.VMEM((CPW, M), jnp.float32),
            pltpu.VMEM((CH, F_ATOM), jnp.float32),
            pltpu.VMEM((CH, F_ATOM), jnp.float32),
            pltpu.VMEM((2 * _GI, _TW), jnp.int32),
            pltpu.SemaphoreType.DMA,
            pltpu.SemaphoreType.DMA,
            pltpu.SemaphoreType.DMA,
            pltpu.SemaphoreType.DMA,
        ],
    )
    return f(tab, adj3, w3)


def _out_body(atom_ref, agg_ref, w1_ref, b1_ref, out_ref):
    x = atom_ref[...] * agg_ref[...]
    y = jnp.dot(x, w1_ref[...], preferred_element_type=jnp.float32)
    out_ref[...] = jnp.maximum(y + b1_ref[...], 0.0)


def _linear_relu(atom2, agg2, W1, b1):
    nb = 1000
    grid = N // nb
    return pl.pallas_call(
        _out_body,
        grid=(grid,),
        in_specs=[
            pl.BlockSpec((nb, F_ATOM), lambda i: (i, 0)),
            pl.BlockSpec((nb, F_ATOM), lambda i: (i, 0)),
            pl.BlockSpec((F_ATOM, F_ATOM), lambda i: (0, 0)),
            pl.BlockSpec((1, F_ATOM), lambda i: (0, 0)),
        ],
        out_specs=pl.BlockSpec((nb, F_ATOM), lambda i: (i, 0)),
        out_shape=jax.ShapeDtypeStruct((N, F_ATOM), jnp.float32),
    )(atom2, agg2, W1, b1.reshape(1, F_ATOM))


@jax.jit
def kernel(atom, bond, adj_matrix, W1, b1):
    atom2 = atom[0]
    bond2 = bond[0].reshape(N, M * F_BOND)
    w, tab = _prep(bond2, atom2)

    pad = ((0, N_PAD - N), (0, 0))
    adj3 = jnp.pad(adj_matrix[0], pad).reshape(NW, NG, _GI)
    w3 = jnp.pad(w, pad).reshape(NW, CPW, M)

    agg = _sc_aggregate(tab, adj3, w3)
    agg2 = agg.reshape(N_PAD, F_ATOM)[:N]

    out = _linear_relu(atom2, agg2, W1, b1)
    return out.reshape(1, N, F_ATOM)
